# baseline (speedup 1.0000x reference)
import functools

import jax
import jax.numpy as jnp
from jax import lax
from jax.experimental import pallas as pl
from jax.experimental.pallas import tpu as pltpu
from jax.experimental.pallas import tpu_sc as plsc

H_IN, W_IN = 8, 16
H_OUT, W_OUT = 6, 12
P = H_OUT * W_OUT
PIX = H_IN * W_IN
L = 16
NC, NS = 2, 16
NW = NC * NS
B = 2048
CH = 32
BT = 256
N_BT = B // BT


V_OFFS = (0, 16, 32, 48, P - L)


def _prep(im_v, jm_v, c00a, w00a, w01a, w10a, w11a):
    for off in V_OFFS:
        im = im_v[pl.ds(off, L)]
        jm = jm_v[pl.ds(off, L)]
        i0 = im.astype(jnp.int32)
        j0 = jm.astype(jnp.int32)
        di = im - i0.astype(jnp.float32)
        dj = jm - j0.astype(jnp.float32)
        w11 = di * dj
        w01 = dj - w11
        w10 = di - w11
        w00 = (1.0 - di) - w01
        c00a[pl.ds(off, L)] = i0 * W_IN + j0
        w00a[pl.ds(off, L)] = w00
        w01a[pl.ds(off, L)] = w01
        w10a[pl.ds(off, L)] = w10
        w11a[pl.ds(off, L)] = w11


def _interp_body(vals_hbm, imap_hbm, jmap_hbm, out_hbm,
                 im_v, jm_v, c00a, w00a, w01a, w10a, w11a,
                 in0, in1, o0, o1, sin0, sin1, so0, so1):
    cp_i = pltpu.make_async_copy(imap_hbm, im_v, sin0)
    cp_j = pltpu.make_async_copy(jmap_hbm, jm_v, sin1)
    cp_i.start()
    cp_j.start()
    cp_i.wait()
    cp_j.wait()
    _prep(im_v, jm_v, c00a, w00a, w01a, w10a, w11a)

    c = lax.axis_index("s") * NC + lax.axis_index("c")
    ibufs = ((in0, sin0), (in1, sin1))
    obufs = ((o0, so0), (o1, so1))

    def in_copy(tb, buf, sem):
        return pltpu.make_async_copy(
            vals_hbm.at[c, :, pl.ds(tb * BT, BT)], buf, sem)

    def out_copy(tb, buf, sem):
        return pltpu.make_async_copy(
            buf, out_hbm.at[:, c, pl.ds(tb * BT, BT)], sem)

    def compute(bi, ob):
        @plsc.parallel_loop(0, P, 1, unroll=4)
        def pbody(p):
            pix = c00a[pl.ds(p, L)][0]
            w00 = w00a[pl.ds(p, L)][0]
            w01 = w01a[pl.ds(p, L)][0]
            w10 = w10a[pl.ds(p, L)][0]
            w11 = w11a[pl.ds(p, L)][0]
            for bb in range(BT // L):
                s = pl.ds(bb * L, L)
                f00 = bi[pix, s]
                f01 = bi[pix + 1, s]
                f10 = bi[pix + W_IN, s]
                f11 = bi[pix + W_IN + 1, s]
                ob[p, s] = f00 * w00 + f01 * w01 + f10 * w10 + f11 * w11

    in_copy(0, in0, sin0).start()
    for tb in range(N_BT):
        bi, si = ibufs[tb % 2]
        in_copy(tb, bi, si).wait()
        if tb + 1 < N_BT:
            nbuf, nsem = ibufs[(tb + 1) % 2]
            in_copy(tb + 1, nbuf, nsem).start()
        ob, so = obufs[tb % 2]
        if tb >= 2:
            out_copy(tb - 2, ob, so).wait()
        compute(bi, ob)
        out_copy(tb, ob, so).start()
    out_copy(N_BT - 2, obufs[(N_BT - 2) % 2][0], obufs[(N_BT - 2) % 2][1]).wait()
    out_copy(N_BT - 1, obufs[(N_BT - 1) % 2][0], obufs[(N_BT - 1) % 2][1]).wait()


@jax.jit
def _interp(vals_t, i_map, j_map):
    mesh = plsc.VectorSubcoreMesh(core_axis_name="c", subcore_axis_name="s")
    f = pl.kernel(
        _interp_body,
        mesh=mesh,
        out_type=jax.ShapeDtypeStruct((P, CH, B), jnp.float32),
        scratch_types=[
            pltpu.VMEM((P,), jnp.float32),
            pltpu.VMEM((P,), jnp.float32),
            pltpu.VMEM((P + L,), jnp.int32),
            pltpu.VMEM((P + L,), jnp.float32),
            pltpu.VMEM((P + L,), jnp.float32),
            pltpu.VMEM((P + L,), jnp.float32),
            pltpu.VMEM((P + L,), jnp.float32),
            pltpu.VMEM((PIX, BT), jnp.float32),
            pltpu.VMEM((PIX, BT), jnp.float32),
            pltpu.VMEM((P, BT), jnp.float32),
            pltpu.VMEM((P, BT), jnp.float32),
            pltpu.SemaphoreType.DMA,
            pltpu.SemaphoreType.DMA,
            pltpu.SemaphoreType.DMA,
            pltpu.SemaphoreType.DMA,
        ],
    )
    return f(vals_t, i_map, j_map)


def kernel(values, i_map, j_map):
    b, c, h, w = values.shape
    vals_t = jnp.transpose(values, (1, 2, 3, 0)).reshape(c, h * w, b)
    out = _interp(vals_t, i_map.reshape(-1), j_map.reshape(-1))
    return jnp.transpose(out.reshape(H_OUT, W_OUT, c, b), (3, 2, 0, 1))


# speedup vs baseline: 1.0591x; 1.0591x over previous
import functools

import jax
import jax.numpy as jnp
from jax import lax
from jax.experimental import pallas as pl
from jax.experimental.pallas import tpu as pltpu
from jax.experimental.pallas import tpu_sc as plsc

H_IN, W_IN = 8, 16
H_OUT, W_OUT = 6, 12
P = H_OUT * W_OUT
PIX = H_IN * W_IN
L = 16
NC, NS = 2, 16
NW = NC * NS
B = 2048
CH = 32
BT = 256
N_BT = B // BT


V_OFFS = (0, 16, 32, 48, P - L)


def _prep(im_v, jm_v, c00a, w00a, w01a, w10a, w11a):
    for off in V_OFFS:
        im = im_v[pl.ds(off, L)]
        jm = jm_v[pl.ds(off, L)]
        i0 = im.astype(jnp.int32)
        j0 = jm.astype(jnp.int32)
        di = im - i0.astype(jnp.float32)
        dj = jm - j0.astype(jnp.float32)
        w11 = di * dj
        w01 = dj - w11
        w10 = di - w11
        w00 = (1.0 - di) - w01
        c00a[pl.ds(off, L)] = i0 * W_IN + j0
        w00a[pl.ds(off, L)] = w00
        w01a[pl.ds(off, L)] = w01
        w10a[pl.ds(off, L)] = w10
        w11a[pl.ds(off, L)] = w11


def _interp_body(vals_hbm, imap_hbm, jmap_hbm, out_hbm,
                 im_v, jm_v, c00a, w00a, w01a, w10a, w11a,
                 in0, in1, o0, o1, sin0, sin1, so0, so1):
    c = lax.axis_index("s") * NC + lax.axis_index("c")
    ibufs = ((in0, sin0), (in1, sin1))
    obufs = ((o0, so0), (o1, so1))

    def in_copy(tb, buf, sem):
        return pltpu.make_async_copy(
            vals_hbm.at[c, :, pl.ds(tb * BT, BT)], buf, sem)

    def out_copy(tb, buf, sem):
        return pltpu.make_async_copy(
            buf, out_hbm.at[:, c, pl.ds(tb * BT, BT)], sem)

    in_copy(0, in0, sin0).start()
    cp_i = pltpu.make_async_copy(imap_hbm, im_v, so0)
    cp_j = pltpu.make_async_copy(jmap_hbm, jm_v, so1)
    cp_i.start()
    cp_j.start()
    cp_i.wait()
    cp_j.wait()
    _prep(im_v, jm_v, c00a, w00a, w01a, w10a, w11a)

    def compute(bi, ob):
        @plsc.parallel_loop(0, P, 1, unroll=2)
        def pbody(p):
            pix = c00a[pl.ds(p, L)][0]
            w00 = w00a[pl.ds(p, L)][0]
            w01 = w01a[pl.ds(p, L)][0]
            w10 = w10a[pl.ds(p, L)][0]
            w11 = w11a[pl.ds(p, L)][0]
            for bb in range(BT // L):
                s = pl.ds(bb * L, L)
                f00 = bi[pix, s]
                f01 = bi[pix + 1, s]
                f10 = bi[pix + W_IN, s]
                f11 = bi[pix + W_IN + 1, s]
                ob[p, s] = f00 * w00 + f01 * w01 + f10 * w10 + f11 * w11

    for tb in range(N_BT):
        bi, si = ibufs[tb % 2]
        in_copy(tb, bi, si).wait()
        if tb + 1 < N_BT:
            nbuf, nsem = ibufs[(tb + 1) % 2]
            in_copy(tb + 1, nbuf, nsem).start()
        ob, so = obufs[tb % 2]
        if tb >= 2:
            out_copy(tb - 2, ob, so).wait()
        compute(bi, ob)
        out_copy(tb, ob, so).start()
    out_copy(N_BT - 2, obufs[(N_BT - 2) % 2][0], obufs[(N_BT - 2) % 2][1]).wait()
    out_copy(N_BT - 1, obufs[(N_BT - 1) % 2][0], obufs[(N_BT - 1) % 2][1]).wait()


@jax.jit
def _interp(vals_t, i_map, j_map):
    mesh = plsc.VectorSubcoreMesh(core_axis_name="c", subcore_axis_name="s")
    f = pl.kernel(
        _interp_body,
        mesh=mesh,
        out_type=jax.ShapeDtypeStruct((P, CH, B), jnp.float32),
        scratch_types=[
            pltpu.VMEM((P,), jnp.float32),
            pltpu.VMEM((P,), jnp.float32),
            pltpu.VMEM((P + L,), jnp.int32),
            pltpu.VMEM((P + L,), jnp.float32),
            pltpu.VMEM((P + L,), jnp.float32),
            pltpu.VMEM((P + L,), jnp.float32),
            pltpu.VMEM((P + L,), jnp.float32),
            pltpu.VMEM((PIX, BT), jnp.float32),
            pltpu.VMEM((PIX, BT), jnp.float32),
            pltpu.VMEM((P, BT), jnp.float32),
            pltpu.VMEM((P, BT), jnp.float32),
            pltpu.SemaphoreType.DMA,
            pltpu.SemaphoreType.DMA,
            pltpu.SemaphoreType.DMA,
            pltpu.SemaphoreType.DMA,
        ],
    )
    return f(vals_t, i_map, j_map)


def kernel(values, i_map, j_map):
    b, c, h, w = values.shape
    vals_t = jnp.transpose(values, (1, 2, 3, 0)).reshape(c, h * w, b)
    out = _interp(vals_t, i_map.reshape(-1), j_map.reshape(-1))
    return jnp.transpose(out.reshape(H_OUT, W_OUT, c, b), (3, 2, 0, 1))
